# Initial kernel scaffold; baseline (speedup 1.0000x reference)
import jax
import jax.numpy as jnp
from jax.experimental import pallas as pl


def kernel(x, edge_index, pos_edge_index, neg_edge_index, W1, b1, W2, b2, score_w, score_b):
    raise NotImplementedError("write your pallas kernel here")


# speedup vs baseline: 10.3068x; 10.3068x over previous
import functools

import jax
import jax.numpy as jnp
from jax import lax
from jax.experimental import pallas as pl
from jax.experimental.pallas import tpu as pltpu
from jax.experimental.pallas import tpu_sc as plsc

N = 10000
E = 320000
EP = 160000
D_IN = 128
D_H = 128
D_OUT = 64

NC = 2
NS = 16
NW = NC * NS

NPAD = 10240
ROWS_PER_TILE = NPAD // NS
CHUNK = 128
NCHUNK = 80
EPAD = NW * NCHUNK * CHUNK
SE_PER_TILE = (2 * EP) // NW

_mesh = functools.partial(
    plsc.VectorSubcoreMesh, core_axis_name="c", subcore_axis_name="s",
    num_cores=NC, num_subcores=NS)


def _wid():
    return lax.axis_index("c") * NS + lax.axis_index("s")


def _deg_body(dst_hbm, deg_out, dstv, ones_v, zb, acc):
    cid = lax.axis_index("c")
    sid = lax.axis_index("s")
    wid = cid * NS + sid

    def zb_init(i, _):
        zb[pl.ds(i * 16, 16)] = jnp.zeros((16,), jnp.float32)
        return 0
    lax.fori_loop(0, ROWS_PER_TILE // 16, zb_init, 0)

    def ones_init(i, _):
        ones_v[pl.ds(i * 16, 16)] = jnp.ones((16,), jnp.float32)
        return 0
    lax.fori_loop(0, CHUNK // 16, ones_init, 0)

    pltpu.sync_copy(zb, acc.at[pl.ds(sid * ROWS_PER_TILE, ROWS_PER_TILE)])
    pltpu.sync_copy(dst_hbm.at[wid], dstv)
    plsc.subcore_barrier()

    def body(j, _):
        pltpu.sync_copy(ones_v, acc.at[dstv.at[j]], add=True)
        return 0
    lax.fori_loop(0, NCHUNK, body, 0)

    plsc.subcore_barrier()
    pltpu.sync_copy(acc.at[pl.ds(sid * ROWS_PER_TILE, ROWS_PER_TILE)],
                    deg_out.at[cid, pl.ds(sid * ROWS_PER_TILE, ROWS_PER_TILE)])


@functools.cache
def _deg_kernel():
    return pl.kernel(
        _deg_body,
        out_type=jax.ShapeDtypeStruct((NC, NPAD), jnp.float32),
        mesh=_mesh(),
        scratch_types=[
            pltpu.VMEM((NCHUNK, CHUNK), jnp.int32),
            pltpu.VMEM((CHUNK,), jnp.float32),
            pltpu.VMEM((ROWS_PER_TILE,), jnp.float32),
            pltpu.VMEM_SHARED((NPAD,), jnp.float32),
        ],
    )


def _msg_body(d, g_hbm, src_hbm, dst_hbm, part_out, srcv, dstv, buf, acc):
    cid = lax.axis_index("c")
    sid = lax.axis_index("s")
    wid = cid * NS + sid

    def buf_init(i, _):
        for c in range(d // 16):
            buf[i, pl.ds(c * 16, 16)] = jnp.zeros((16,), jnp.float32)
        return 0
    lax.fori_loop(0, CHUNK, buf_init, 0)

    for k in range(ROWS_PER_TILE // CHUNK):
        pltpu.sync_copy(
            buf, acc.at[pl.ds(sid * ROWS_PER_TILE + k * CHUNK, CHUNK)])

    pltpu.sync_copy(src_hbm.at[wid], srcv)
    pltpu.sync_copy(dst_hbm.at[wid], dstv)
    plsc.subcore_barrier()

    def body(j, _):
        pltpu.sync_copy(g_hbm.at[srcv.at[j]], buf)
        pltpu.sync_copy(buf, acc.at[dstv.at[j]], add=True)
        return 0
    lax.fori_loop(0, NCHUNK, body, 0)

    plsc.subcore_barrier()
    pltpu.sync_copy(acc.at[pl.ds(sid * ROWS_PER_TILE, ROWS_PER_TILE)],
                    part_out.at[cid, pl.ds(sid * ROWS_PER_TILE, ROWS_PER_TILE)])


@functools.cache
def _make_msg_kernel(d):
    return pl.kernel(
        functools.partial(_msg_body, d),
        out_type=jax.ShapeDtypeStruct((NC, NPAD, d), jnp.float32),
        mesh=_mesh(),
        scratch_types=[
            pltpu.VMEM((NCHUNK, CHUNK), jnp.int32),
            pltpu.VMEM((NCHUNK, CHUNK), jnp.int32),
            pltpu.VMEM((CHUNK, d), jnp.float32),
            pltpu.VMEM_SHARED((NPAD, d), jnp.float32),
        ],
        compiler_params=pltpu.CompilerParams(
            use_tc_tiling_on_sc=(d % 128 == 0)),
    )


def _score_body(s_hbm, src_hbm, dst_hbm, sb_hbm, out_hbm, loss_out,
                s_v, srcv, dstv, outv, sbv, lpv):
    wid = _wid()

    pltpu.sync_copy(s_hbm, s_v)
    pltpu.sync_copy(src_hbm.at[wid], srcv)
    pltpu.sync_copy(dst_hbm.at[wid], dstv)
    pltpu.sync_copy(sb_hbm, sbv)
    sb = sbv[...]

    def body(i, acc):
        sv = srcv[pl.ds(i * 16, 16)]
        dv = dstv[pl.ds(i * 16, 16)]
        a = plsc.load_gather(s_v, [sv])
        b = plsc.load_gather(s_v, [dv])
        dist = a - b
        outv[pl.ds(i * 16, 16)] = jnp.maximum(dist + sb, 0.0)
        return acc + dist

    acc = lax.fori_loop(0, SE_PER_TILE // 16, body,
                        jnp.zeros((16,), jnp.float32))
    lpv[...] = acc
    pltpu.sync_copy(outv, out_hbm.at[wid])
    pltpu.sync_copy(lpv, loss_out.at[wid])


@functools.cache
def _score_kernel():
    return pl.kernel(
        _score_body,
        out_type=(
            jax.ShapeDtypeStruct((NW, SE_PER_TILE), jnp.float32),
            jax.ShapeDtypeStruct((NW, 16), jnp.float32),
        ),
        mesh=_mesh(),
        scratch_types=[
            pltpu.VMEM((NPAD,), jnp.float32),
            pltpu.VMEM((SE_PER_TILE,), jnp.int32),
            pltpu.VMEM((SE_PER_TILE,), jnp.int32),
            pltpu.VMEM((SE_PER_TILE,), jnp.float32),
            pltpu.VMEM((16,), jnp.float32),
            pltpu.VMEM((16,), jnp.float32),
        ],
        compiler_params=pltpu.CompilerParams(needs_layout_passes=False),
    )


TC_B = 1024
TC_GRID = NPAD // TC_B


def _mm1_body(x_ref, w_ref, da_ref, db_ref, g_ref, dinv_ref):
    deg = da_ref[...] + db_ref[...] + 1.0
    dinv = lax.rsqrt(jnp.maximum(deg, 1.0))
    h = jnp.dot(x_ref[...], w_ref[...], preferred_element_type=jnp.float32, precision=lax.Precision.HIGHEST)
    g_ref[...] = h * dinv
    dinv_ref[...] = dinv


def _mm1(x_pad, w1, dega, degb):
    return pl.pallas_call(
        _mm1_body,
        grid=(TC_GRID,),
        in_specs=[
            pl.BlockSpec((TC_B, D_IN), lambda i: (i, 0)),
            pl.BlockSpec((D_IN, D_H), lambda i: (0, 0)),
            pl.BlockSpec((TC_B, 1), lambda i: (i, 0)),
            pl.BlockSpec((TC_B, 1), lambda i: (i, 0)),
        ],
        out_specs=[
            pl.BlockSpec((TC_B, D_H), lambda i: (i, 0)),
            pl.BlockSpec((TC_B, 1), lambda i: (i, 0)),
        ],
        out_shape=[
            jax.ShapeDtypeStruct((NPAD, D_H), jnp.float32),
            jax.ShapeDtypeStruct((NPAD, 1), jnp.float32),
        ],
    )(x_pad, w1, dega, degb)


def _mm2_body(a0_ref, a1_ref, g1_ref, dinv_ref, w2_ref, b1_ref, g2_ref):
    dinv = dinv_ref[...]
    out1 = jnp.maximum(
        dinv * (a0_ref[...] + a1_ref[...] + g1_ref[...]) + b1_ref[...], 0.0)
    g2_ref[...] = dinv * jnp.dot(out1, w2_ref[...],
                                 preferred_element_type=jnp.float32, precision=lax.Precision.HIGHEST)


def _mm2(a0, a1, g1, dinv, w2, b1):
    return pl.pallas_call(
        _mm2_body,
        grid=(TC_GRID,),
        in_specs=[
            pl.BlockSpec((TC_B, D_H), lambda i: (i, 0)),
            pl.BlockSpec((TC_B, D_H), lambda i: (i, 0)),
            pl.BlockSpec((TC_B, D_H), lambda i: (i, 0)),
            pl.BlockSpec((TC_B, 1), lambda i: (i, 0)),
            pl.BlockSpec((D_H, D_OUT), lambda i: (0, 0)),
            pl.BlockSpec((1, D_H), lambda i: (0, 0)),
        ],
        out_specs=pl.BlockSpec((TC_B, D_OUT), lambda i: (i, 0)),
        out_shape=jax.ShapeDtypeStruct((NPAD, D_OUT), jnp.float32),
    )(a0, a1, g1, dinv, w2, b1)


def _mm3_body(a0_ref, a1_ref, g2_ref, dinv_ref, b2_ref, w_ref, s_ref):
    h2 = dinv_ref[...] * (a0_ref[...] + a1_ref[...] + g2_ref[...]) + b2_ref[...]
    s_ref[...] = jnp.dot(h2, w_ref[...], preferred_element_type=jnp.float32, precision=lax.Precision.HIGHEST)


def _mm3(a0, a1, g2, dinv, b2, w):
    return pl.pallas_call(
        _mm3_body,
        grid=(TC_GRID,),
        in_specs=[
            pl.BlockSpec((TC_B, D_OUT), lambda i: (i, 0)),
            pl.BlockSpec((TC_B, D_OUT), lambda i: (i, 0)),
            pl.BlockSpec((TC_B, D_OUT), lambda i: (i, 0)),
            pl.BlockSpec((TC_B, 1), lambda i: (i, 0)),
            pl.BlockSpec((1, D_OUT), lambda i: (0, 0)),
            pl.BlockSpec((D_OUT, 1), lambda i: (0, 0)),
        ],
        out_specs=pl.BlockSpec((TC_B, 1), lambda i: (i, 0)),
        out_shape=jax.ShapeDtypeStruct((NPAD, 1), jnp.float32),
    )(a0, a1, g2, dinv, b2, w)


def kernel(x, edge_index, pos_edge_index, neg_edge_index,
           W1, b1, W2, b2, score_w, score_b):
    x_pad = jnp.zeros((NPAD, D_IN), jnp.float32).at[:N].set(x)

    ei = edge_index.astype(jnp.int32)
    pad_e = EPAD - E
    src = jnp.concatenate([ei[0], jnp.zeros((pad_e,), jnp.int32)])
    dst = jnp.concatenate([ei[1], jnp.full((pad_e,), N, jnp.int32)])
    src_idx = src.reshape(NW, NCHUNK, CHUNK)
    dst_idx = dst.reshape(NW, NCHUNK, CHUNK)

    te = jnp.concatenate([pos_edge_index, neg_edge_index],
                         axis=-1).astype(jnp.int32)
    te_src = te[0].reshape(NW, SE_PER_TILE)
    te_dst = te[1].reshape(NW, SE_PER_TILE)

    sb16 = jnp.broadcast_to(score_b.astype(jnp.float32), (16,))

    deg_part = _deg_kernel()(dst_idx)
    dega = deg_part[0].reshape(NPAD, 1)
    degb = deg_part[1].reshape(NPAD, 1)

    g1, dinv = _mm1(x_pad, W1, dega, degb)
    part1 = _make_msg_kernel(D_H)(g1, src_idx, dst_idx)

    g2 = _mm2(part1[0], part1[1], g1, dinv, W2, b1.reshape(1, D_H))
    part2 = _make_msg_kernel(D_OUT)(g2, src_idx, dst_idx)

    s = _mm3(part2[0], part2[1], g2, dinv, b2.reshape(1, D_OUT),
             score_w.reshape(D_OUT, 1))
    s_flat = s.reshape(NPAD)

    out_tiles, loss_part = _score_kernel()(s_flat, te_src, te_dst, sb16)
    out = out_tiles.reshape(2 * EP)
    score_loss = jnp.sum(loss_part) / (2.0 * EP)
    return (out, score_loss)


# speedup vs baseline: 11.0337x; 1.0705x over previous
import functools

import jax
import jax.numpy as jnp
from jax import lax
from jax.experimental import pallas as pl
from jax.experimental.pallas import tpu as pltpu
from jax.experimental.pallas import tpu_sc as plsc

N = 10000
E = 320000
EP = 160000
D_IN = 128
D_H = 128
D_OUT = 64

NC = 2
NS = 16
NW = NC * NS

NPAD = 10240
ROWS_PER_TILE = NPAD // NS
CHUNK = 128
NCHUNK = 80
EPAD = NW * NCHUNK * CHUNK
NPHASE = 2
CPP = NCHUNK // NPHASE
SE_PER_TILE = (2 * EP) // NW

_mesh = functools.partial(
    plsc.VectorSubcoreMesh, core_axis_name="c", subcore_axis_name="s",
    num_cores=NC, num_subcores=NS)


def _wid():
    return lax.axis_index("c") * NS + lax.axis_index("s")


def _deg_body(dst_hbm, deg_out, dstv, ones_v, zb, acc):
    cid = lax.axis_index("c")
    sid = lax.axis_index("s")
    wid = cid * NS + sid

    def zb_init(i, _):
        zb[pl.ds(i * 16, 16)] = jnp.zeros((16,), jnp.float32)
        return 0
    lax.fori_loop(0, ROWS_PER_TILE // 16, zb_init, 0)

    def ones_init(i, _):
        ones_v[pl.ds(i * 16, 16)] = jnp.ones((16,), jnp.float32)
        return 0
    lax.fori_loop(0, CHUNK // 16, ones_init, 0)

    pltpu.sync_copy(zb, acc.at[pl.ds(sid * ROWS_PER_TILE, ROWS_PER_TILE)])
    pltpu.sync_copy(dst_hbm.at[wid], dstv)
    plsc.subcore_barrier()

    def body(j, _):
        pltpu.sync_copy(ones_v, acc.at[dstv.at[j]], add=True)
        return 0
    lax.fori_loop(0, NCHUNK, body, 0)

    plsc.subcore_barrier()
    pltpu.sync_copy(acc.at[pl.ds(sid * ROWS_PER_TILE, ROWS_PER_TILE)],
                    deg_out.at[cid, pl.ds(sid * ROWS_PER_TILE, ROWS_PER_TILE)])


@functools.cache
def _deg_kernel():
    return pl.kernel(
        _deg_body,
        out_type=jax.ShapeDtypeStruct((NC, NPAD), jnp.float32),
        mesh=_mesh(),
        scratch_types=[
            pltpu.VMEM((NCHUNK, CHUNK), jnp.int32),
            pltpu.VMEM((CHUNK,), jnp.float32),
            pltpu.VMEM((ROWS_PER_TILE,), jnp.float32),
            pltpu.VMEM_SHARED((NPAD,), jnp.float32),
        ],
    )


def _msg_body(d, g_hbm, src_hbm, dst_hbm, part_out, srcv, dstv, buf0, buf1,
              gs0, gs1, acc):
    cid = lax.axis_index("c")
    sid = lax.axis_index("s")
    wid = cid * NS + sid

    def buf_init(i, _):
        for c in range(d // 16):
            buf0[i, pl.ds(c * 16, 16)] = jnp.zeros((16,), jnp.float32)
        return 0
    lax.fori_loop(0, CHUNK, buf_init, 0)

    for k in range(ROWS_PER_TILE // CHUNK):
        pltpu.sync_copy(
            buf0, acc.at[pl.ds(sid * ROWS_PER_TILE + k * CHUNK, CHUNK)])

    plsc.subcore_barrier()

    for p in range(NPHASE):
        pltpu.sync_copy(src_hbm.at[wid, pl.ds(p * CPP, CPP)], srcv)
        pltpu.sync_copy(dst_hbm.at[wid, pl.ds(p * CPP, CPP)], dstv)
        pltpu.make_async_copy(g_hbm.at[srcv.at[0]], buf0, gs0).start()

        def body(j2, _):
            j = j2 * 2
            pltpu.make_async_copy(g_hbm.at[srcv.at[j]], buf0, gs0).wait()
            pltpu.make_async_copy(g_hbm.at[srcv.at[j + 1]], buf1, gs1).start()
            pltpu.sync_copy(buf0, acc.at[dstv.at[j]], add=True)
            pltpu.make_async_copy(g_hbm.at[srcv.at[j + 1]], buf1, gs1).wait()

            @pl.when(j + 2 < CPP)
            def _():
                pltpu.make_async_copy(
                    g_hbm.at[srcv.at[j + 2]], buf0, gs0).start()

            pltpu.sync_copy(buf1, acc.at[dstv.at[j + 1]], add=True)
            return 0
        lax.fori_loop(0, CPP // 2, body, 0)

    plsc.subcore_barrier()
    pltpu.sync_copy(acc.at[pl.ds(sid * ROWS_PER_TILE, ROWS_PER_TILE)],
                    part_out.at[cid, pl.ds(sid * ROWS_PER_TILE, ROWS_PER_TILE)])


@functools.cache
def _make_msg_kernel(d):
    return pl.kernel(
        functools.partial(_msg_body, d),
        out_type=jax.ShapeDtypeStruct((NC, NPAD, d), jnp.float32),
        mesh=_mesh(),
        scratch_types=[
            pltpu.VMEM((CPP, CHUNK), jnp.int32),
            pltpu.VMEM((CPP, CHUNK), jnp.int32),
            pltpu.VMEM((CHUNK, d), jnp.float32),
            pltpu.VMEM((CHUNK, d), jnp.float32),
            pltpu.SemaphoreType.DMA,
            pltpu.SemaphoreType.DMA,
            pltpu.VMEM_SHARED((NPAD, d), jnp.float32),
        ],
        compiler_params=pltpu.CompilerParams(
            use_tc_tiling_on_sc=(d % 128 == 0)),
    )


def _score_body(s_hbm, src_hbm, dst_hbm, sb_hbm, out_hbm, loss_out,
                s_v, srcv, dstv, outv, sbv, lpv):
    wid = _wid()

    pltpu.sync_copy(s_hbm, s_v)
    pltpu.sync_copy(src_hbm.at[wid], srcv)
    pltpu.sync_copy(dst_hbm.at[wid], dstv)
    pltpu.sync_copy(sb_hbm, sbv)
    sb = sbv[...]

    def body(i, acc):
        sv = srcv[pl.ds(i * 16, 16)]
        dv = dstv[pl.ds(i * 16, 16)]
        a = plsc.load_gather(s_v, [sv])
        b = plsc.load_gather(s_v, [dv])
        dist = a - b
        outv[pl.ds(i * 16, 16)] = jnp.maximum(dist + sb, 0.0)
        return acc + dist

    acc = lax.fori_loop(0, SE_PER_TILE // 16, body,
                        jnp.zeros((16,), jnp.float32))
    lpv[...] = acc
    pltpu.sync_copy(outv, out_hbm.at[wid])
    pltpu.sync_copy(lpv, loss_out.at[wid])


@functools.cache
def _score_kernel():
    return pl.kernel(
        _score_body,
        out_type=(
            jax.ShapeDtypeStruct((NW, SE_PER_TILE), jnp.float32),
            jax.ShapeDtypeStruct((NW, 16), jnp.float32),
        ),
        mesh=_mesh(),
        scratch_types=[
            pltpu.VMEM((NPAD,), jnp.float32),
            pltpu.VMEM((SE_PER_TILE,), jnp.int32),
            pltpu.VMEM((SE_PER_TILE,), jnp.int32),
            pltpu.VMEM((SE_PER_TILE,), jnp.float32),
            pltpu.VMEM((16,), jnp.float32),
            pltpu.VMEM((16,), jnp.float32),
        ],
        compiler_params=pltpu.CompilerParams(needs_layout_passes=False),
    )


TC_B = 1024
TC_GRID = NPAD // TC_B


def _mm1_body(x_ref, w_ref, da_ref, db_ref, g_ref, dinv_ref):
    deg = da_ref[...] + db_ref[...] + 1.0
    dinv = lax.rsqrt(jnp.maximum(deg, 1.0))
    h = jnp.dot(x_ref[...], w_ref[...], preferred_element_type=jnp.float32, precision=lax.Precision.HIGHEST)
    g_ref[...] = h * dinv
    dinv_ref[...] = dinv


def _mm1(x_pad, w1, dega, degb):
    return pl.pallas_call(
        _mm1_body,
        grid=(TC_GRID,),
        in_specs=[
            pl.BlockSpec((TC_B, D_IN), lambda i: (i, 0)),
            pl.BlockSpec((D_IN, D_H), lambda i: (0, 0)),
            pl.BlockSpec((TC_B, 1), lambda i: (i, 0)),
            pl.BlockSpec((TC_B, 1), lambda i: (i, 0)),
        ],
        out_specs=[
            pl.BlockSpec((TC_B, D_H), lambda i: (i, 0)),
            pl.BlockSpec((TC_B, 1), lambda i: (i, 0)),
        ],
        out_shape=[
            jax.ShapeDtypeStruct((NPAD, D_H), jnp.float32),
            jax.ShapeDtypeStruct((NPAD, 1), jnp.float32),
        ],
    )(x_pad, w1, dega, degb)


def _mm2_body(a0_ref, a1_ref, g1_ref, dinv_ref, w2_ref, b1_ref, g2_ref):
    dinv = dinv_ref[...]
    out1 = jnp.maximum(
        dinv * (a0_ref[...] + a1_ref[...] + g1_ref[...]) + b1_ref[...], 0.0)
    g2_ref[...] = dinv * jnp.dot(out1, w2_ref[...],
                                 preferred_element_type=jnp.float32, precision=lax.Precision.HIGHEST)


def _mm2(a0, a1, g1, dinv, w2, b1):
    return pl.pallas_call(
        _mm2_body,
        grid=(TC_GRID,),
        in_specs=[
            pl.BlockSpec((TC_B, D_H), lambda i: (i, 0)),
            pl.BlockSpec((TC_B, D_H), lambda i: (i, 0)),
            pl.BlockSpec((TC_B, D_H), lambda i: (i, 0)),
            pl.BlockSpec((TC_B, 1), lambda i: (i, 0)),
            pl.BlockSpec((D_H, D_OUT), lambda i: (0, 0)),
            pl.BlockSpec((1, D_H), lambda i: (0, 0)),
        ],
        out_specs=pl.BlockSpec((TC_B, D_OUT), lambda i: (i, 0)),
        out_shape=jax.ShapeDtypeStruct((NPAD, D_OUT), jnp.float32),
    )(a0, a1, g1, dinv, w2, b1)


def _mm3_body(a0_ref, a1_ref, g2_ref, dinv_ref, b2_ref, w_ref, s_ref):
    h2 = dinv_ref[...] * (a0_ref[...] + a1_ref[...] + g2_ref[...]) + b2_ref[...]
    s_ref[...] = jnp.dot(h2, w_ref[...], preferred_element_type=jnp.float32, precision=lax.Precision.HIGHEST)


def _mm3(a0, a1, g2, dinv, b2, w):
    return pl.pallas_call(
        _mm3_body,
        grid=(TC_GRID,),
        in_specs=[
            pl.BlockSpec((TC_B, D_OUT), lambda i: (i, 0)),
            pl.BlockSpec((TC_B, D_OUT), lambda i: (i, 0)),
            pl.BlockSpec((TC_B, D_OUT), lambda i: (i, 0)),
            pl.BlockSpec((TC_B, 1), lambda i: (i, 0)),
            pl.BlockSpec((1, D_OUT), lambda i: (0, 0)),
            pl.BlockSpec((D_OUT, 1), lambda i: (0, 0)),
        ],
        out_specs=pl.BlockSpec((TC_B, 1), lambda i: (i, 0)),
        out_shape=jax.ShapeDtypeStruct((NPAD, 1), jnp.float32),
    )(a0, a1, g2, dinv, b2, w)


def kernel(x, edge_index, pos_edge_index, neg_edge_index,
           W1, b1, W2, b2, score_w, score_b):
    x_pad = jnp.zeros((NPAD, D_IN), jnp.float32).at[:N].set(x)

    ei = edge_index.astype(jnp.int32)
    pad_e = EPAD - E
    src = jnp.concatenate([ei[0], jnp.zeros((pad_e,), jnp.int32)])
    dst = jnp.concatenate([ei[1], jnp.full((pad_e,), N, jnp.int32)])
    src_idx = src.reshape(NW, NCHUNK, CHUNK)
    dst_idx = dst.reshape(NW, NCHUNK, CHUNK)

    te = jnp.concatenate([pos_edge_index, neg_edge_index],
                         axis=-1).astype(jnp.int32)
    te_src = te[0].reshape(NW, SE_PER_TILE)
    te_dst = te[1].reshape(NW, SE_PER_TILE)

    sb16 = jnp.broadcast_to(score_b.astype(jnp.float32), (16,))

    deg_part = _deg_kernel()(dst_idx)
    dega = deg_part[0].reshape(NPAD, 1)
    degb = deg_part[1].reshape(NPAD, 1)

    g1, dinv = _mm1(x_pad, W1, dega, degb)
    part1 = _make_msg_kernel(D_H)(g1, src_idx, dst_idx)

    g2 = _mm2(part1[0], part1[1], g1, dinv, W2, b1.reshape(1, D_H))
    part2 = _make_msg_kernel(D_OUT)(g2, src_idx, dst_idx)

    s = _mm3(part2[0], part2[1], g2, dinv, b2.reshape(1, D_OUT),
             score_w.reshape(D_OUT, 1))
    s_flat = s.reshape(NPAD)

    out_tiles, loss_part = _score_kernel()(s_flat, te_src, te_dst, sb16)
    out = out_tiles.reshape(2 * EP)
    score_loss = jnp.sum(loss_part) / (2.0 * EP)
    return (out, score_loss)
